# baseline (speedup 1.0000x reference)
import functools

import jax
import jax.numpy as jnp
from jax import lax
from jax.experimental import pallas as pl
from jax.experimental.pallas import tpu as pltpu
from jax.experimental.pallas import tpu_sc as plsc

NC = 2
NS = 16
NW = NC * NS
CK = 64
DEGW = 128
SPLIT_NUM = 29
SPLIT_DEN = 40


def _build_sc_kernels(NP, NGRP, G, D):
    mesh = plsc.VectorSubcoreMesh(core_axis_name="c", subcore_axis_name="s")
    RPW = NP // NS
    A = (NGRP * SPLIT_NUM) // SPLIT_DEN

    @functools.partial(
        pl.kernel, mesh=mesh,
        out_type=jax.ShapeDtypeStruct((NC, NP, DEGW), jnp.float32),
        scratch_types=[
            pltpu.VMEM((G, 2, CK), jnp.int32),
            pltpu.VMEM((CK, DEGW), jnp.float32),
            pltpu.VMEM_SHARED((NP, DEGW), jnp.float32),
        ],
    )
    def deg_kernel(idx_hbm, ones_hbm, zeros_hbm, out_hbm, idx_v, ones_v, acc):
        c = lax.axis_index("c")
        s = lax.axis_index("s")
        base = s * RPW
        half = NGRP // NC
        lo = c * half
        hi = half + c * (NGRP - half)
        pltpu.sync_copy(zeros_hbm.at[pl.ds(base, RPW)], acc.at[pl.ds(base, RPW)])
        pltpu.sync_copy(ones_hbm, ones_v)
        plsc.subcore_barrier()

        @pl.loop(0, hi - lo)
        def _(gg):
            pltpu.sync_copy(idx_hbm.at[s, lo + gg], idx_v)

            @pl.loop(0, G)
            def _(j):
                pltpu.sync_copy(ones_v, acc.at[idx_v.at[j, 1]], add=True)

        plsc.subcore_barrier()
        pltpu.sync_copy(acc.at[pl.ds(base, RPW)], out_hbm.at[c, pl.ds(base, RPW)])

    @functools.partial(
        pl.kernel, mesh=mesh,
        out_type=jax.ShapeDtypeStruct((NC, NP, D), jnp.float32),
        scratch_types=[
            pltpu.VMEM((G, 2, CK), jnp.int32),
            pltpu.VMEM((CK, D), jnp.float32),
            pltpu.VMEM((CK, D), jnp.float32),
            pltpu.VMEM_SHARED((NP, D), jnp.float32),
            pltpu.SemaphoreType.DMA,
            pltpu.SemaphoreType.DMA,
        ],
    )
    def msg_kernel(h_hbm, idx_hbm, zeros_hbm, out_hbm,
                   idx_v, buf_a, buf_b, acc, sem_a, sem_b):
        c = lax.axis_index("c")
        s = lax.axis_index("s")
        base = s * RPW
        lo = c * A
        hi = A + c * (NGRP - A)
        pltpu.sync_copy(zeros_hbm.at[pl.ds(base, RPW)], acc.at[pl.ds(base, RPW)])
        plsc.subcore_barrier()

        @pl.loop(0, hi - lo)
        def _(gg):
            pltpu.sync_copy(idx_hbm.at[s, lo + gg], idx_v)
            pltpu.async_copy(h_hbm.at[idx_v.at[0, 0]], buf_a, sem_a)

            @pl.loop(0, G, step=2)
            def _(j):
                pltpu.make_async_copy(h_hbm.at[idx_v.at[j, 0]], buf_a,
                                      sem_a).wait()
                pltpu.async_copy(h_hbm.at[idx_v.at[j + 1, 0]], buf_b, sem_b)
                pltpu.sync_copy(buf_a, acc.at[idx_v.at[j, 1]], add=True)
                pltpu.make_async_copy(h_hbm.at[idx_v.at[j + 1, 0]], buf_b,
                                      sem_b).wait()

                @pl.when(j + 2 < G)
                def _():
                    pltpu.async_copy(h_hbm.at[idx_v.at[j + 2, 0]], buf_a, sem_a)

                pltpu.sync_copy(buf_b, acc.at[idx_v.at[j + 1, 1]], add=True)

        plsc.subcore_barrier()
        pltpu.sync_copy(acc.at[pl.ds(base, RPW)], out_hbm.at[c, pl.ds(base, RPW)])

    return deg_kernel, msg_kernel


def _prep_call(NP, Nn, Dh):
    def body(x_ref, w_ref, degp_ref, h_ref, dinv_ref):
        deg = degp_ref[0, :, 0:1] + degp_ref[1, :, 0:1]
        row = lax.broadcasted_iota(jnp.int32, (NP, 1), 0)
        dinv = jnp.where((deg > 0.0) & (row < Nn),
                         lax.rsqrt(jnp.maximum(deg, 1e-12)), 0.0)
        h = jnp.dot(x_ref[...], w_ref[...], preferred_element_type=jnp.float32)
        h_ref[...] = h * dinv
        dinv_ref[...] = dinv

    return pl.pallas_call(
        body,
        out_shape=(jax.ShapeDtypeStruct((NP, Dh), jnp.float32),
                   jax.ShapeDtypeStruct((NP, 1), jnp.float32)),
    )


def _mid_call(NP, Dh, Do):
    def body(p_ref, dinv_ref, b1_ref, w2_ref, h2_ref):
        dinv = dinv_ref[...]
        agg = p_ref[0] + p_ref[1]
        h = jnp.maximum(agg * dinv + b1_ref[...], 0.0)
        h2_ref[...] = jnp.dot(h, w2_ref[...],
                              preferred_element_type=jnp.float32) * dinv

    return pl.pallas_call(
        body,
        out_shape=jax.ShapeDtypeStruct((NP, Dh), jnp.float32),
    )


def _final_call(NP, Nn, Do):
    def body(p_ref, dinv_ref, b2_ref, o_ref):
        agg = p_ref[0, :Nn, :] + p_ref[1, :Nn, :]
        o_ref[...] = agg * dinv_ref[:Nn, :] + b2_ref[...]

    return pl.pallas_call(
        body,
        out_shape=jax.ShapeDtypeStruct((Nn, Do), jnp.float32),
    )


def kernel(x, edge_index, W1, b1, W2, b2):
    Nn, Din = x.shape
    Dh = W1.shape[1]
    Do = W2.shape[1]
    E = edge_index.shape[1]
    G = 16
    NGRP = -(-E // (NS * G * CK * SPLIT_DEN)) * SPLIT_DEN
    EP = NGRP * NS * G * CK
    NP = -(-(Nn + 1) // (NS * 8)) * (NS * 8)

    ei = edge_index.astype(jnp.int32)
    pad = jnp.full((EP - E,), Nn, jnp.int32)
    src = jnp.concatenate([ei[0], pad]).reshape(NS, NGRP, G, 1, CK)
    dst = jnp.concatenate([ei[1], pad]).reshape(NS, NGRP, G, 1, CK)
    idx = jnp.concatenate([src, dst], axis=3)
    x_pad = jnp.pad(x, ((0, NP - Nn), (0, 0)))

    ones_deg = jnp.ones((CK, DEGW), jnp.float32)
    zeros_deg = jnp.zeros((NP, DEGW), jnp.float32)
    zeros_d = jnp.zeros((NP, Dh), jnp.float32)

    deg_k, msg_k = _build_sc_kernels(NP, NGRP, G, Dh)

    degp = deg_k(idx, ones_deg, zeros_deg)
    h1p, dinv = _prep_call(NP, Nn, Dh)(x_pad, W1, degp)
    p1 = msg_k(h1p, idx, zeros_d)
    h2p = _mid_call(NP, Dh, Do)(p1, dinv, b1.reshape(1, Dh), W2)
    p2 = msg_k(h2p, idx, zeros_d)
    return _final_call(NP, Nn, Do)(p2, dinv, b2.reshape(1, Do))


# speedup vs baseline: 36.9588x; 36.9588x over previous
import functools

import jax
import jax.numpy as jnp
from jax import lax
from jax.experimental import pallas as pl
from jax.experimental.pallas import tpu as pltpu
from jax.experimental.pallas import tpu_sc as plsc

NC = 2
NS = 16
NW = NC * NS
CK = 64
DEGW = 128
SPLIT_NUM = 15
SPLIT_DEN = 20


def _build_sc_kernels(NP, NGRP, G, D):
    mesh = plsc.VectorSubcoreMesh(core_axis_name="c", subcore_axis_name="s")
    RPW = NP // NS
    A = (NGRP * SPLIT_NUM) // SPLIT_DEN

    @functools.partial(
        pl.kernel, mesh=mesh,
        out_type=jax.ShapeDtypeStruct((NC, NP, DEGW), jnp.float32),
        scratch_types=[
            pltpu.VMEM((G, 2, CK), jnp.int32),
            pltpu.VMEM((CK, DEGW), jnp.float32),
            pltpu.VMEM_SHARED((NP, DEGW), jnp.float32),
        ],
    )
    def deg_kernel(idx_hbm, ones_hbm, zeros_hbm, out_hbm, idx_v, ones_v, acc):
        c = lax.axis_index("c")
        s = lax.axis_index("s")
        base = s * RPW
        half = NGRP // NC
        lo = c * half
        hi = half + c * (NGRP - half)
        pltpu.sync_copy(zeros_hbm.at[pl.ds(base, RPW)], acc.at[pl.ds(base, RPW)])
        pltpu.sync_copy(ones_hbm, ones_v)
        plsc.subcore_barrier()

        @pl.loop(0, hi - lo)
        def _(gg):
            pltpu.sync_copy(idx_hbm.at[s, lo + gg], idx_v)

            @pl.loop(0, G)
            def _(j):
                pltpu.sync_copy(ones_v, acc.at[idx_v.at[j, 1]], add=True)

        plsc.subcore_barrier()
        pltpu.sync_copy(acc.at[pl.ds(base, RPW)], out_hbm.at[c, pl.ds(base, RPW)])

    @functools.partial(
        pl.kernel, mesh=mesh,
        out_type=jax.ShapeDtypeStruct((NC, NP, D), jnp.float32),
        scratch_types=[
            pltpu.VMEM((G, 2, CK), jnp.int32),
            pltpu.VMEM((CK, D), jnp.float32),
            pltpu.VMEM((CK, D), jnp.float32),
            pltpu.VMEM_SHARED((NP, D), jnp.float32),
            pltpu.SemaphoreType.DMA,
            pltpu.SemaphoreType.DMA,
        ],
    )
    def msg_kernel(h_hbm, idx_hbm, zeros_hbm, out_hbm,
                   idx_v, buf_a, buf_b, acc, sem_a, sem_b):
        c = lax.axis_index("c")
        s = lax.axis_index("s")
        base = s * RPW
        lo = c * A
        hi = A + c * (NGRP - A)
        pltpu.sync_copy(zeros_hbm.at[pl.ds(base, RPW)], acc.at[pl.ds(base, RPW)])
        plsc.subcore_barrier()

        @pl.loop(0, hi - lo)
        def _(gg):
            pltpu.sync_copy(idx_hbm.at[s, lo + gg], idx_v)
            pltpu.async_copy(h_hbm.at[idx_v.at[0, 0]], buf_a, sem_a)

            @pl.loop(0, G, step=2)
            def _(j):
                pltpu.make_async_copy(h_hbm.at[idx_v.at[j, 0]], buf_a,
                                      sem_a).wait()
                pltpu.async_copy(h_hbm.at[idx_v.at[j + 1, 0]], buf_b, sem_b)
                pltpu.sync_copy(buf_a, acc.at[idx_v.at[j, 1]], add=True)
                pltpu.make_async_copy(h_hbm.at[idx_v.at[j + 1, 0]], buf_b,
                                      sem_b).wait()

                @pl.when(j + 2 < G)
                def _():
                    pltpu.async_copy(h_hbm.at[idx_v.at[j + 2, 0]], buf_a, sem_a)

                pltpu.sync_copy(buf_b, acc.at[idx_v.at[j + 1, 1]], add=True)

        plsc.subcore_barrier()
        pltpu.sync_copy(acc.at[pl.ds(base, RPW)], out_hbm.at[c, pl.ds(base, RPW)])

    return deg_kernel, msg_kernel


def _prep_call(NP, Nn, Dh):
    def body(x_ref, w_ref, degp_ref, h_ref, dinv_ref):
        deg = degp_ref[0, :, 0:1] + degp_ref[1, :, 0:1]
        row = lax.broadcasted_iota(jnp.int32, (NP, 1), 0)
        dinv = jnp.where((deg > 0.0) & (row < Nn),
                         lax.rsqrt(jnp.maximum(deg, 1e-12)), 0.0)
        h = jnp.dot(x_ref[...], w_ref[...], preferred_element_type=jnp.float32)
        h_ref[...] = h * dinv
        dinv_ref[...] = dinv

    return pl.pallas_call(
        body,
        out_shape=(jax.ShapeDtypeStruct((NP, Dh), jnp.float32),
                   jax.ShapeDtypeStruct((NP, 1), jnp.float32)),
    )


def _mid_call(NP, Dh, Do):
    def body(p_ref, dinv_ref, b1_ref, w2_ref, h2_ref):
        dinv = dinv_ref[...]
        agg = p_ref[0] + p_ref[1]
        h = jnp.maximum(agg * dinv + b1_ref[...], 0.0)
        h2_ref[...] = jnp.dot(h, w2_ref[...],
                              preferred_element_type=jnp.float32) * dinv

    return pl.pallas_call(
        body,
        out_shape=jax.ShapeDtypeStruct((NP, Dh), jnp.float32),
    )


def _final_call(NP, Nn, Do):
    def body(p_ref, dinv_ref, b2_ref, o_ref):
        agg = p_ref[0, :Nn, :] + p_ref[1, :Nn, :]
        o_ref[...] = agg * dinv_ref[:Nn, :] + b2_ref[...]

    return pl.pallas_call(
        body,
        out_shape=jax.ShapeDtypeStruct((Nn, Do), jnp.float32),
    )


def kernel(x, edge_index, W1, b1, W2, b2):
    Nn, Din = x.shape
    Dh = W1.shape[1]
    Do = W2.shape[1]
    E = edge_index.shape[1]
    G = 16
    NGRP = -(-E // (NS * G * CK * SPLIT_DEN)) * SPLIT_DEN
    EP = NGRP * NS * G * CK
    NP = -(-(Nn + 1) // (NS * 8)) * (NS * 8)

    ei = edge_index.astype(jnp.int32)
    pad = Nn + jnp.arange(EP - E, dtype=jnp.int32) % (NP - Nn)
    src = jnp.concatenate([ei[0], pad]).reshape(NS, NGRP, G, 1, CK)
    dst = jnp.concatenate([ei[1], pad]).reshape(NS, NGRP, G, 1, CK)
    idx = jnp.concatenate([src, dst], axis=3)
    x_pad = jnp.pad(x, ((0, NP - Nn), (0, 0)))

    ones_deg = jnp.ones((CK, DEGW), jnp.float32)
    zeros_deg = jnp.zeros((NP, DEGW), jnp.float32)
    zeros_d = jnp.zeros((NP, Dh), jnp.float32)

    deg_k, msg_k = _build_sc_kernels(NP, NGRP, G, Dh)

    degp = deg_k(idx, ones_deg, zeros_deg)
    h1p, dinv = _prep_call(NP, Nn, Dh)(x_pad, W1, degp)
    p1 = msg_k(h1p, idx, zeros_d)
    h2p = _mid_call(NP, Dh, Do)(p1, dinv, b1.reshape(1, Dh), W2)
    p2 = msg_k(h2p, idx, zeros_d)
    return _final_call(NP, Nn, Do)(p2, dinv, b2.reshape(1, Do))


# speedup vs baseline: 48.4119x; 1.3099x over previous
import functools

import jax
import jax.numpy as jnp
from jax import lax
from jax.experimental import pallas as pl
from jax.experimental.pallas import tpu as pltpu
from jax.experimental.pallas import tpu_sc as plsc

NC = 2
NS = 16
NW = NC * NS
CK = 64
DEGW = 128
SPLIT_NUM = 10
SPLIT_DEN = 20


def _build_sc_kernels(NP, NGRP, G, D):
    mesh = plsc.VectorSubcoreMesh(core_axis_name="c", subcore_axis_name="s")
    RPW = NP // NS
    A = (NGRP * SPLIT_NUM) // SPLIT_DEN

    @functools.partial(
        pl.kernel, mesh=mesh,
        out_type=jax.ShapeDtypeStruct((NC, NP, DEGW), jnp.float32),
        scratch_types=[
            pltpu.VMEM((G, 2, CK), jnp.int32),
            pltpu.VMEM((CK, DEGW), jnp.float32),
            pltpu.VMEM_SHARED((NP, DEGW), jnp.float32),
        ],
    )
    def deg_kernel(idx_hbm, ones_hbm, zeros_hbm, out_hbm, idx_v, ones_v, acc):
        c = lax.axis_index("c")
        s = lax.axis_index("s")
        base = s * RPW
        half = NGRP // NC
        lo = c * half
        hi = half + c * (NGRP - half)
        pltpu.sync_copy(zeros_hbm.at[pl.ds(base, RPW)], acc.at[pl.ds(base, RPW)])
        pltpu.sync_copy(ones_hbm, ones_v)
        plsc.subcore_barrier()

        @pl.loop(0, hi - lo)
        def _(gg):
            pltpu.sync_copy(idx_hbm.at[s, lo + gg], idx_v)

            @pl.loop(0, G)
            def _(j):
                pltpu.sync_copy(ones_v, acc.at[idx_v.at[j, 1]], add=True)

        plsc.subcore_barrier()
        pltpu.sync_copy(acc.at[pl.ds(base, RPW)], out_hbm.at[c, pl.ds(base, RPW)])

    @functools.partial(
        pl.kernel, mesh=mesh,
        out_type=jax.ShapeDtypeStruct((NC, NP, D), jnp.float32),
        scratch_types=[
            pltpu.VMEM((G, 2, CK), jnp.int32),
            pltpu.VMEM((CK, D), jnp.float32),
            pltpu.VMEM((CK, D), jnp.float32),
            pltpu.VMEM_SHARED((NP, D), jnp.float32),
            pltpu.SemaphoreType.DMA,
            pltpu.SemaphoreType.DMA,
        ],
    )
    def msg_kernel(h_hbm, idx_hbm, zeros_hbm, out_hbm,
                   idx_v, buf_a, buf_b, acc, sem_a, sem_b):
        c = lax.axis_index("c")
        s = lax.axis_index("s")
        base = s * RPW
        lo = c * A
        hi = A + c * (NGRP - A)
        pltpu.sync_copy(zeros_hbm.at[pl.ds(base, RPW)], acc.at[pl.ds(base, RPW)])
        plsc.subcore_barrier()

        @pl.loop(0, hi - lo)
        def _(gg):
            pltpu.sync_copy(idx_hbm.at[s, lo + gg], idx_v)
            pltpu.async_copy(h_hbm.at[idx_v.at[0, 0]], buf_a, sem_a)

            @pl.loop(0, G, step=2)
            def _(j):
                pltpu.make_async_copy(h_hbm.at[idx_v.at[j, 0]], buf_a,
                                      sem_a).wait()
                pltpu.async_copy(h_hbm.at[idx_v.at[j + 1, 0]], buf_b, sem_b)
                pltpu.sync_copy(buf_a, acc.at[idx_v.at[j, 1]], add=True)
                pltpu.make_async_copy(h_hbm.at[idx_v.at[j + 1, 0]], buf_b,
                                      sem_b).wait()

                @pl.when(j + 2 < G)
                def _():
                    pltpu.async_copy(h_hbm.at[idx_v.at[j + 2, 0]], buf_a, sem_a)

                pltpu.sync_copy(buf_b, acc.at[idx_v.at[j + 1, 1]], add=True)

        plsc.subcore_barrier()
        pltpu.sync_copy(acc.at[pl.ds(base, RPW)], out_hbm.at[c, pl.ds(base, RPW)])

    return deg_kernel, msg_kernel


def _prep_call(NP, Nn, Dh):
    def body(x_ref, w_ref, degp_ref, h_ref, dinv_ref):
        deg = degp_ref[0, :, 0:1] + degp_ref[1, :, 0:1]
        row = lax.broadcasted_iota(jnp.int32, (NP, 1), 0)
        dinv = jnp.where((deg > 0.0) & (row < Nn),
                         lax.rsqrt(jnp.maximum(deg, 1e-12)), 0.0)
        h = jnp.dot(x_ref[...], w_ref[...], preferred_element_type=jnp.float32)
        h_ref[...] = h * dinv
        dinv_ref[...] = dinv

    return pl.pallas_call(
        body,
        out_shape=(jax.ShapeDtypeStruct((NP, Dh), jnp.float32),
                   jax.ShapeDtypeStruct((NP, 1), jnp.float32)),
    )


def _mid_call(NP, Dh, Do):
    def body(p_ref, dinv_ref, b1_ref, w2_ref, h2_ref):
        dinv = dinv_ref[...]
        agg = p_ref[0] + p_ref[1]
        h = jnp.maximum(agg * dinv + b1_ref[...], 0.0)
        h2_ref[...] = jnp.dot(h, w2_ref[...],
                              preferred_element_type=jnp.float32) * dinv

    return pl.pallas_call(
        body,
        out_shape=jax.ShapeDtypeStruct((NP, Dh), jnp.float32),
    )


def _final_call(NP, Nn, Do):
    def body(p_ref, dinv_ref, b2_ref, o_ref):
        agg = p_ref[0, :Nn, :] + p_ref[1, :Nn, :]
        o_ref[...] = agg * dinv_ref[:Nn, :] + b2_ref[...]

    return pl.pallas_call(
        body,
        out_shape=jax.ShapeDtypeStruct((Nn, Do), jnp.float32),
    )


def kernel(x, edge_index, W1, b1, W2, b2):
    Nn, Din = x.shape
    Dh = W1.shape[1]
    Do = W2.shape[1]
    E = edge_index.shape[1]
    G = 16
    NGRP = -(-E // (NS * G * CK * SPLIT_DEN)) * SPLIT_DEN
    EP = NGRP * NS * G * CK
    NP = -(-(Nn + 1) // (NS * 8)) * (NS * 8)

    ei = edge_index.astype(jnp.int32)
    pad = Nn + jnp.arange(EP - E, dtype=jnp.int32) % (NP - Nn)
    src = jnp.concatenate([ei[0], pad]).reshape(NS, NGRP, G, 1, CK)
    dst = jnp.concatenate([ei[1], pad]).reshape(NS, NGRP, G, 1, CK)
    idx = jnp.concatenate([src, dst], axis=3)
    x_pad = jnp.pad(x, ((0, NP - Nn), (0, 0)))

    ones_deg = jnp.ones((CK, DEGW), jnp.float32)
    zeros_deg = jnp.zeros((NP, DEGW), jnp.float32)
    zeros_d = jnp.zeros((NP, Dh), jnp.float32)

    deg_k, msg_k = _build_sc_kernels(NP, NGRP, G, Dh)

    degp = deg_k(idx, ones_deg, zeros_deg)
    h1p, dinv = _prep_call(NP, Nn, Dh)(x_pad, W1, degp)
    p1 = msg_k(h1p, idx, zeros_d)
    h2p = _mid_call(NP, Dh, Do)(p1, dinv, b1.reshape(1, Dh), W2)
    p2 = msg_k(h2p, idx, zeros_d)
    return _final_call(NP, Nn, Do)(p2, dinv, b2.reshape(1, Do))


# speedup vs baseline: 54.6689x; 1.1292x over previous
import functools

import jax
import jax.numpy as jnp
from jax import lax
from jax.experimental import pallas as pl
from jax.experimental.pallas import tpu as pltpu
from jax.experimental.pallas import tpu_sc as plsc

NC = 2
NS = 16
NW = NC * NS
CK = 96
DEGW = 128
SPLIT_NUM = 10
SPLIT_DEN = 20


def _build_sc_kernels(NP, NGRP, G, D):
    mesh = plsc.VectorSubcoreMesh(core_axis_name="c", subcore_axis_name="s")
    RPW = NP // NS
    A = (NGRP * SPLIT_NUM) // SPLIT_DEN

    @functools.partial(
        pl.kernel, mesh=mesh,
        out_type=jax.ShapeDtypeStruct((NC, NP, DEGW), jnp.float32),
        scratch_types=[
            pltpu.VMEM((G, 2, CK), jnp.int32),
            pltpu.VMEM((CK, DEGW), jnp.float32),
            pltpu.VMEM_SHARED((NP, DEGW), jnp.float32),
        ],
    )
    def deg_kernel(idx_hbm, ones_hbm, zeros_hbm, out_hbm, idx_v, ones_v, acc):
        c = lax.axis_index("c")
        s = lax.axis_index("s")
        base = s * RPW
        half = NGRP // NC
        lo = c * half
        hi = half + c * (NGRP - half)
        pltpu.sync_copy(zeros_hbm.at[pl.ds(base, RPW)], acc.at[pl.ds(base, RPW)])
        pltpu.sync_copy(ones_hbm, ones_v)
        plsc.subcore_barrier()

        @pl.loop(0, hi - lo)
        def _(gg):
            pltpu.sync_copy(idx_hbm.at[s, lo + gg], idx_v)

            @pl.loop(0, G)
            def _(j):
                pltpu.sync_copy(ones_v, acc.at[idx_v.at[j, 1]], add=True)

        plsc.subcore_barrier()
        pltpu.sync_copy(acc.at[pl.ds(base, RPW)], out_hbm.at[c, pl.ds(base, RPW)])

    @functools.partial(
        pl.kernel, mesh=mesh,
        out_type=jax.ShapeDtypeStruct((NC, NP, D), jnp.float32),
        scratch_types=[
            pltpu.VMEM((G, 2, CK), jnp.int32),
            pltpu.VMEM((CK, D), jnp.float32),
            pltpu.VMEM((CK, D), jnp.float32),
            pltpu.VMEM_SHARED((NP, D), jnp.float32),
            pltpu.SemaphoreType.DMA,
            pltpu.SemaphoreType.DMA,
        ],
    )
    def msg_kernel(h_hbm, idx_hbm, zeros_hbm, out_hbm,
                   idx_v, buf_a, buf_b, acc, sem_a, sem_b):
        c = lax.axis_index("c")
        s = lax.axis_index("s")
        base = s * RPW
        lo = c * A
        hi = A + c * (NGRP - A)
        pltpu.sync_copy(zeros_hbm.at[pl.ds(base, RPW)], acc.at[pl.ds(base, RPW)])
        plsc.subcore_barrier()

        @pl.loop(0, hi - lo)
        def _(gg):
            pltpu.sync_copy(idx_hbm.at[s, lo + gg], idx_v)
            pltpu.async_copy(h_hbm.at[idx_v.at[0, 0]], buf_a, sem_a)

            @pl.loop(0, G, step=2)
            def _(j):
                pltpu.make_async_copy(h_hbm.at[idx_v.at[j, 0]], buf_a,
                                      sem_a).wait()
                pltpu.async_copy(h_hbm.at[idx_v.at[j + 1, 0]], buf_b, sem_b)
                pltpu.sync_copy(buf_a, acc.at[idx_v.at[j, 1]], add=True)
                pltpu.make_async_copy(h_hbm.at[idx_v.at[j + 1, 0]], buf_b,
                                      sem_b).wait()

                @pl.when(j + 2 < G)
                def _():
                    pltpu.async_copy(h_hbm.at[idx_v.at[j + 2, 0]], buf_a, sem_a)

                pltpu.sync_copy(buf_b, acc.at[idx_v.at[j + 1, 1]], add=True)

        plsc.subcore_barrier()
        pltpu.sync_copy(acc.at[pl.ds(base, RPW)], out_hbm.at[c, pl.ds(base, RPW)])

    return deg_kernel, msg_kernel


def _prep_call(NP, Nn, Dh):
    def body(x_ref, w_ref, degp_ref, h_ref, dinv_ref):
        deg = degp_ref[0, :, 0:1] + degp_ref[1, :, 0:1]
        row = lax.broadcasted_iota(jnp.int32, (NP, 1), 0)
        dinv = jnp.where((deg > 0.0) & (row < Nn),
                         lax.rsqrt(jnp.maximum(deg, 1e-12)), 0.0)
        h = jnp.dot(x_ref[...], w_ref[...], preferred_element_type=jnp.float32)
        h_ref[...] = h * dinv
        dinv_ref[...] = dinv

    return pl.pallas_call(
        body,
        out_shape=(jax.ShapeDtypeStruct((NP, Dh), jnp.float32),
                   jax.ShapeDtypeStruct((NP, 1), jnp.float32)),
    )


def _mid_call(NP, Dh, Do):
    def body(p_ref, dinv_ref, b1_ref, w2_ref, h2_ref):
        dinv = dinv_ref[...]
        agg = p_ref[0] + p_ref[1]
        h = jnp.maximum(agg * dinv + b1_ref[...], 0.0)
        h2_ref[...] = jnp.dot(h, w2_ref[...],
                              preferred_element_type=jnp.float32) * dinv

    return pl.pallas_call(
        body,
        out_shape=jax.ShapeDtypeStruct((NP, Dh), jnp.float32),
    )


def _final_call(NP, Nn, Do):
    def body(p_ref, dinv_ref, b2_ref, o_ref):
        agg = p_ref[0, :Nn, :] + p_ref[1, :Nn, :]
        o_ref[...] = agg * dinv_ref[:Nn, :] + b2_ref[...]

    return pl.pallas_call(
        body,
        out_shape=jax.ShapeDtypeStruct((Nn, Do), jnp.float32),
    )


def kernel(x, edge_index, W1, b1, W2, b2):
    Nn, Din = x.shape
    Dh = W1.shape[1]
    Do = W2.shape[1]
    E = edge_index.shape[1]
    G = 16
    NGRP = -(-E // (NS * G * CK))
    EP = NGRP * NS * G * CK
    NP = -(-(Nn + 1) // (NS * 8)) * (NS * 8)

    ei = edge_index.astype(jnp.int32)
    pad = Nn + jnp.arange(EP - E, dtype=jnp.int32) % (NP - Nn)
    src = jnp.concatenate([ei[0], pad]).reshape(NS, NGRP, G, 1, CK)
    dst = jnp.concatenate([ei[1], pad]).reshape(NS, NGRP, G, 1, CK)
    idx = jnp.concatenate([src, dst], axis=3)
    x_pad = jnp.pad(x, ((0, NP - Nn), (0, 0)))

    ones_deg = jnp.ones((CK, DEGW), jnp.float32)
    zeros_deg = jnp.zeros((NP, DEGW), jnp.float32)
    zeros_d = jnp.zeros((NP, Dh), jnp.float32)

    deg_k, msg_k = _build_sc_kernels(NP, NGRP, G, Dh)

    degp = deg_k(idx, ones_deg, zeros_deg)
    h1p, dinv = _prep_call(NP, Nn, Dh)(x_pad, W1, degp)
    p1 = msg_k(h1p, idx, zeros_d)
    h2p = _mid_call(NP, Dh, Do)(p1, dinv, b1.reshape(1, Dh), W2)
    p2 = msg_k(h2p, idx, zeros_d)
    return _final_call(NP, Nn, Do)(p2, dinv, b2.reshape(1, Do))


# speedup vs baseline: 56.0477x; 1.0252x over previous
import functools

import jax
import jax.numpy as jnp
from jax import lax
from jax.experimental import pallas as pl
from jax.experimental.pallas import tpu as pltpu
from jax.experimental.pallas import tpu_sc as plsc

NC = 2
NS = 16
NW = NC * NS
CK = 112
DEGW = 128
SPLIT_NUM = 10
SPLIT_DEN = 20


def _build_sc_kernels(NP, NGRP, G, D):
    mesh = plsc.VectorSubcoreMesh(core_axis_name="c", subcore_axis_name="s")
    RPW = NP // NS
    A = (NGRP * SPLIT_NUM) // SPLIT_DEN

    @functools.partial(
        pl.kernel, mesh=mesh,
        out_type=jax.ShapeDtypeStruct((NC, NP, DEGW), jnp.float32),
        scratch_types=[
            pltpu.VMEM((G, 2, CK), jnp.int32),
            pltpu.VMEM((CK, DEGW), jnp.float32),
            pltpu.VMEM_SHARED((NP, DEGW), jnp.float32),
        ],
    )
    def deg_kernel(idx_hbm, ones_hbm, zeros_hbm, out_hbm, idx_v, ones_v, acc):
        c = lax.axis_index("c")
        s = lax.axis_index("s")
        base = s * RPW
        half = NGRP // NC
        lo = c * half
        hi = half + c * (NGRP - half)
        pltpu.sync_copy(zeros_hbm.at[pl.ds(base, RPW)], acc.at[pl.ds(base, RPW)])
        pltpu.sync_copy(ones_hbm, ones_v)
        plsc.subcore_barrier()

        @pl.loop(0, hi - lo)
        def _(gg):
            pltpu.sync_copy(idx_hbm.at[s, lo + gg], idx_v)

            @pl.loop(0, G)
            def _(j):
                pltpu.sync_copy(ones_v, acc.at[idx_v.at[j, 1]], add=True)

        plsc.subcore_barrier()
        pltpu.sync_copy(acc.at[pl.ds(base, RPW)], out_hbm.at[c, pl.ds(base, RPW)])

    @functools.partial(
        pl.kernel, mesh=mesh,
        out_type=jax.ShapeDtypeStruct((NC, NP, D), jnp.float32),
        scratch_types=[
            pltpu.VMEM((G, 2, CK), jnp.int32),
            pltpu.VMEM((CK, D), jnp.float32),
            pltpu.VMEM((CK, D), jnp.float32),
            pltpu.VMEM_SHARED((NP, D), jnp.float32),
            pltpu.SemaphoreType.DMA,
            pltpu.SemaphoreType.DMA,
        ],
    )
    def msg_kernel(h_hbm, idx_hbm, zeros_hbm, out_hbm,
                   idx_v, buf_a, buf_b, acc, sem_a, sem_b):
        c = lax.axis_index("c")
        s = lax.axis_index("s")
        base = s * RPW
        lo = c * A
        hi = A + c * (NGRP - A)
        pltpu.sync_copy(zeros_hbm.at[pl.ds(base, RPW)], acc.at[pl.ds(base, RPW)])
        plsc.subcore_barrier()

        @pl.loop(0, hi - lo)
        def _(gg):
            pltpu.sync_copy(idx_hbm.at[s, lo + gg], idx_v)
            pltpu.async_copy(h_hbm.at[idx_v.at[0, 0]], buf_a, sem_a)

            @pl.loop(0, G, step=2)
            def _(j):
                pltpu.make_async_copy(h_hbm.at[idx_v.at[j, 0]], buf_a,
                                      sem_a).wait()
                pltpu.async_copy(h_hbm.at[idx_v.at[j + 1, 0]], buf_b, sem_b)
                pltpu.sync_copy(buf_a, acc.at[idx_v.at[j, 1]], add=True)
                pltpu.make_async_copy(h_hbm.at[idx_v.at[j + 1, 0]], buf_b,
                                      sem_b).wait()

                @pl.when(j + 2 < G)
                def _():
                    pltpu.async_copy(h_hbm.at[idx_v.at[j + 2, 0]], buf_a, sem_a)

                pltpu.sync_copy(buf_b, acc.at[idx_v.at[j + 1, 1]], add=True)

        plsc.subcore_barrier()
        pltpu.sync_copy(acc.at[pl.ds(base, RPW)], out_hbm.at[c, pl.ds(base, RPW)])

    return deg_kernel, msg_kernel


def _prep_call(NP, Nn, Dh):
    def body(x_ref, w_ref, degp_ref, h_ref, dinv_ref):
        deg = degp_ref[0, :, 0:1] + degp_ref[1, :, 0:1]
        row = lax.broadcasted_iota(jnp.int32, (NP, 1), 0)
        dinv = jnp.where((deg > 0.0) & (row < Nn),
                         lax.rsqrt(jnp.maximum(deg, 1e-12)), 0.0)
        h = jnp.dot(x_ref[...], w_ref[...], preferred_element_type=jnp.float32)
        h_ref[...] = h * dinv
        dinv_ref[...] = dinv

    return pl.pallas_call(
        body,
        out_shape=(jax.ShapeDtypeStruct((NP, Dh), jnp.float32),
                   jax.ShapeDtypeStruct((NP, 1), jnp.float32)),
    )


def _mid_call(NP, Dh, Do):
    def body(p_ref, dinv_ref, b1_ref, w2_ref, h2_ref):
        dinv = dinv_ref[...]
        agg = p_ref[0] + p_ref[1]
        h = jnp.maximum(agg * dinv + b1_ref[...], 0.0)
        h2_ref[...] = jnp.dot(h, w2_ref[...],
                              preferred_element_type=jnp.float32) * dinv

    return pl.pallas_call(
        body,
        out_shape=jax.ShapeDtypeStruct((NP, Dh), jnp.float32),
    )


def _final_call(NP, Nn, Do):
    def body(p_ref, dinv_ref, b2_ref, o_ref):
        agg = p_ref[0, :Nn, :] + p_ref[1, :Nn, :]
        o_ref[...] = agg * dinv_ref[:Nn, :] + b2_ref[...]

    return pl.pallas_call(
        body,
        out_shape=jax.ShapeDtypeStruct((Nn, Do), jnp.float32),
    )


def kernel(x, edge_index, W1, b1, W2, b2):
    Nn, Din = x.shape
    Dh = W1.shape[1]
    Do = W2.shape[1]
    E = edge_index.shape[1]
    G = 8
    NGRP = -(-E // (NS * G * CK))
    EP = NGRP * NS * G * CK
    NP = -(-(Nn + 1) // (NS * 8)) * (NS * 8)

    ei = edge_index.astype(jnp.int32)
    pad = Nn + jnp.arange(EP - E, dtype=jnp.int32) % (NP - Nn)
    src = jnp.concatenate([ei[0], pad]).reshape(NS, NGRP, G, 1, CK)
    dst = jnp.concatenate([ei[1], pad]).reshape(NS, NGRP, G, 1, CK)
    idx = jnp.concatenate([src, dst], axis=3)
    x_pad = jnp.pad(x, ((0, NP - Nn), (0, 0)))

    ones_deg = jnp.ones((CK, DEGW), jnp.float32)
    zeros_deg = jnp.zeros((NP, DEGW), jnp.float32)
    zeros_d = jnp.zeros((NP, Dh), jnp.float32)

    deg_k, msg_k = _build_sc_kernels(NP, NGRP, G, Dh)

    degp = deg_k(idx, ones_deg, zeros_deg)
    h1p, dinv = _prep_call(NP, Nn, Dh)(x_pad, W1, degp)
    p1 = msg_k(h1p, idx, zeros_d)
    h2p = _mid_call(NP, Dh, Do)(p1, dinv, b1.reshape(1, Dh), W2)
    p2 = msg_k(h2p, idx, zeros_d)
    return _final_call(NP, Nn, Do)(p2, dinv, b2.reshape(1, Do))
